# Initial kernel scaffold; baseline (speedup 1.0000x reference)
import jax
import jax.numpy as jnp
from jax.experimental import pallas as pl


def kernel(x, params, edge_index, batch):
    raise NotImplementedError("write your pallas kernel here")


# speedup vs baseline: 39.5282x; 39.5282x over previous
import functools

import jax
import jax.numpy as jnp
from jax import lax
from jax.experimental import pallas as pl
from jax.experimental.pallas import tpu as pltpu
from jax.experimental.pallas import tpu_sc as plsc

N = 10000
NB = 32
ROWS = 313
NPAD = NB * ROWS
C = 512
E_TOT = 320000 + N
EPAD = E_TOT + 2 * C
F32 = jnp.float32
I32 = jnp.int32

def _sc_edge_body(xw_hbm, as_hbm, ad_hbm, src_hbm, dl_hbm, offs_hbm, zero_hbm,
                  zero16_hbm, num_hbm, den_hbm,
                  num_b, den_b, ad_b, idx_b, dlv_b, xwr_b, asr_b, offs_b,
                  gsem):
    cid = lax.axis_index("c")
    sid = lax.axis_index("s")
    wid = sid * 2 + cid
    base = wid * ROWS

    pltpu.sync_copy(zero_hbm, num_b)
    pltpu.sync_copy(zero16_hbm, den_b)
    pltpu.sync_copy(ad_hbm.at[pl.ds(base, ROWS), :], ad_b)
    pltpu.sync_copy(offs_hbm, offs_b)

    ov = offs_b[pl.ds(wid, 16)]
    start = ov[0]
    end = ov[1]
    astart = (start // 8) * 8
    nchunks = (end - astart + C - 1) // C

    def chunk_body(i, carry):
        cbase = astart + i * C
        pltpu.sync_copy(src_hbm.at[pl.ds(cbase, C)], idx_b)
        pltpu.sync_copy(dl_hbm.at[pl.ds(cbase, C)], dlv_b.at[pl.ds(0, C)])
        cps = []
        for j in range(C // 128):
            sl = pl.ds(j * 128, 128)
            cps.append(pltpu.async_copy(
                xw_hbm.at[idx_b.at[sl]], xwr_b.at[sl, :], gsem))
            cps.append(pltpu.async_copy(
                as_hbm.at[idx_b.at[sl]], asr_b.at[sl, :], gsem))
        for cp in cps:
            cp.wait()

        e0 = jnp.maximum(start - cbase, 0)
        e1 = jnp.minimum(end - cbase, C)

        def edge_body(e, carry2):
            dl = dlv_b[pl.ds(e, 16)][0]
            al = asr_b[e] + ad_b[dl]
            al = jnp.where(al >= 0.0, al, al * 0.2)
            w = jnp.exp(al)
            plsc.addupdate(den_b.at[dl], w)
            for h in range(8):
                wv = lax.broadcast_in_dim(w[h], (16,), ())
                plsc.addupdate(num_b.at[dl, pl.ds(h * 16, 16)],
                               xwr_b[e, pl.ds(h * 16, 16)] * wv)
            return carry2

        lax.fori_loop(e0, e1, edge_body, 0)
        return carry

    lax.fori_loop(0, nchunks, chunk_body, 0)

    pltpu.sync_copy(num_b, num_hbm.at[pl.ds(base, ROWS), :])
    pltpu.sync_copy(den_b, den_hbm.at[pl.ds(base, ROWS), :])


@functools.cache
def _make_sc_edge():
    mesh = plsc.VectorSubcoreMesh(
        core_axis_name="c", subcore_axis_name="s", num_cores=2,
        num_subcores=16)
    return pl.kernel(
        _sc_edge_body,
        out_type=[jax.ShapeDtypeStruct((NPAD, 128), F32),
                  jax.ShapeDtypeStruct((NPAD, 16), F32)],
        mesh=mesh,
        compiler_params=pltpu.CompilerParams(use_tc_tiling_on_sc=False),
        scratch_types=[
        pltpu.VMEM((ROWS, 128), F32),
        pltpu.VMEM((ROWS, 16), F32),
        pltpu.VMEM((ROWS, 16), F32),
        pltpu.VMEM((C,), I32),
        pltpu.VMEM((C + 16,), I32),
        pltpu.VMEM((C, 128), F32),
        pltpu.VMEM((C, 16), F32),
        pltpu.VMEM((48,), I32),
            pltpu.SemaphoreType.DMA,
        ],
    )


def _sc_edge(*args):
    return _make_sc_edge()(*args)


def _bn_x(y, g, b, eps=1e-5):
    m = jnp.mean(y, axis=0)
    v = jnp.var(y, axis=0)
    return g * (y - m) / jnp.sqrt(v + eps) + b


def _ln_x(y, g, b, eps=1e-5):
    m = jnp.mean(y, axis=-1, keepdims=True)
    v = jnp.var(y, axis=-1, keepdims=True)
    return g * (y - m) / jnp.sqrt(v + eps) + b


def _mm_in(a_ref, w_ref, o_ref):
    o_ref[...] = jnp.dot(a_ref[...], w_ref[...], preferred_element_type=F32)


def _mm_proj(a_ref, w_ref, o_ref):
    o_ref[0:N, :] = jnp.dot(a_ref[...], w_ref[...],
                            preferred_element_type=F32)


def kernel(x, params, edge_index, batch):
    p = params

    sl = jnp.arange(N, dtype=edge_index.dtype)
    src = jnp.concatenate([edge_index[0], sl])
    dst = jnp.concatenate([edge_index[1], sl])
    bkt = dst // ROWS
    order = jnp.argsort(bkt)
    src_s = src[order].astype(I32)
    dst_s = dst[order]
    dl_s = (dst_s - (dst_s // ROWS) * ROWS).astype(I32)
    bs = bkt[order].astype(I32)
    offs = jnp.searchsorted(bs, jnp.arange(NB, dtype=I32)).astype(I32)
    offs = jnp.concatenate([offs, jnp.full((48 - NB,), E_TOT, I32)])
    src_pad = jnp.pad(src_s, (0, EPAD - E_TOT))
    dl_pad = jnp.pad(dl_s, (0, EPAD - E_TOT))
    zeros_blk = jnp.zeros((ROWS, 128), F32)
    zeros16 = jnp.zeros((ROWS, 16), F32)

    sds = jax.ShapeDtypeStruct
    mm_in = pl.pallas_call(_mm_in, out_shape=sds((N, 128), F32))
    mm_proj = pl.pallas_call(_mm_proj, out_shape=sds((NPAD, 128), F32))

    y0 = mm_in(x, p["W_in"]) + p["b_in"]
    h = jax.nn.relu(_bn_x(y0, p["bn_in_g"], p["bn_in_b"]))

    for i in range(3):
        g = p["gat%d" % i]
        hr = h
        xw_pad = mm_proj(h, g["W"])
        xw3 = xw_pad[:N].reshape(N, 8, 16)
        a_s = jnp.sum(xw3 * g["att_src"], axis=-1)
        a_d = jnp.sum(xw3 * g["att_dst"], axis=-1)
        as_t = jnp.pad(a_s, ((0, NPAD - N), (0, 8)))
        ad_t = jnp.pad(a_d, ((0, NPAD - N), (0, 8)))
        num, den = _sc_edge(xw_pad, as_t, ad_t, src_pad, dl_pad, offs,
                            zeros_blk, zeros16)
        out3 = num[:N].reshape(N, 8, 16) / den[:N, :8][:, :, None]
        out = out3.reshape(N, 128) if i < 2 else jnp.mean(out3, axis=1)
        h2 = out + g["bias"]
        h2 = _bn_x(h2, g["bn_g"], g["bn_b"])
        h2 = _ln_x(h2, g["ln_g"], g["ln_b"])
        h2 = jax.nn.relu(h2)
        if h2.shape[1] == hr.shape[1]:
            h2 = h2 + hr
        h = h2

    cnt = jax.ops.segment_sum(jnp.ones((N,), F32), batch, num_segments=64)
    h_mean = jax.ops.segment_sum(h, batch, num_segments=64) / jnp.maximum(
        cnt, 1.0)[:, None]
    h_max = jax.ops.segment_max(h, batch, num_segments=64)
    hp = jnp.concatenate([h_mean, h_max], axis=1)
    hp = hp @ p["W_p"] + p["b_p"]
    hp = _bn_x(hp, p["bn_p_g"], p["bn_p_b"])
    hp = jax.nn.relu(hp)
    e = hp @ p["W_e"] + p["b_e"]
    e = _bn_x(e, p["bn_e_g"], p["bn_e_b"])
    e = _ln_x(e, p["ln_e_g"], p["ln_e_b"])
    return jax.nn.relu(e)
